# baseline (speedup 1.0000x reference)
import functools

import jax
import jax.numpy as jnp
from jax import lax
from jax.experimental import pallas as pl
from jax.experimental.pallas import tpu as pltpu
from jax.experimental.pallas import tpu_sc as plsc

N = 10000
E = 320000
NC, NS = 2, 16
NW = NC * NS
CH_E = 64
EPW = 10240
RPW = 640
NP = NS * RPW
DUMMY = N
IBLK = 40
NBUF = 4
RB = 1024


def _mesh():
    return plsc.VectorSubcoreMesh(core_axis_name="c", subcore_axis_name="s")


def _sc_scatter(D):
    @functools.partial(
        pl.kernel,
        out_type=jax.ShapeDtypeStruct((NC, NP, D), jnp.float32),
        mesh=_mesh(),
        compiler_params=pltpu.CompilerParams(use_tc_tiling_on_sc=False),
        scratch_types=[
            pltpu.VMEM_SHARED((NP, D), jnp.float32),
        ],
    )
    def k(g_hbm, srcw_hbm, dstw_hbm, zrow_hbm, out_hbm, acc):
        pl.run_scoped(
            functools.partial(_scatter_body, g_hbm, srcw_hbm, dstw_hbm,
                              zrow_hbm, out_hbm, acc),
            pltpu.VMEM((IBLK, CH_E), jnp.int32),
            pltpu.VMEM((IBLK, CH_E), jnp.int32),
            [pltpu.VMEM((CH_E, D), jnp.float32) for _ in range(NBUF)],
            [pltpu.SemaphoreType.DMA for _ in range(NBUF)],
            [pltpu.SemaphoreType.DMA for _ in range(NBUF)],
        )

    return k


def _scatter_body(g_hbm, srcw_hbm, dstw_hbm, zrow_hbm, out_hbm, acc,
                  src_v, dst_v, bufs, gsems, ssems):
    chunks = EPW // CH_E
    c = lax.axis_index("c")
    s = lax.axis_index("s")
    w = s * NC + c
    pltpu.sync_copy(zrow_hbm, bufs[0])

    @pl.loop(0, RPW // CH_E)
    def _zero(i):
        pltpu.sync_copy(bufs[0], acc.at[pl.ds(s * RPW + i * CH_E, CH_E)])

    plsc.subcore_barrier()

    @pl.loop(0, chunks // IBLK)
    def _block(kb):
        base = w * chunks + kb * IBLK
        pltpu.sync_copy(srcw_hbm.at[pl.ds(base, IBLK)], src_v)
        pltpu.sync_copy(dstw_hbm.at[pl.ds(base, IBLK)], dst_v)
        for r in range(NBUF):
            pltpu.async_copy(g_hbm.at[src_v.at[r]], bufs[r], gsems[r])

        @pl.loop(0, IBLK, step=NBUF)
        def _chunk(j):
            for r in range(NBUF):
                cur = j + r
                pltpu.make_async_copy(g_hbm.at[src_v.at[cur]], bufs[r],
                                      gsems[r]).wait()
                pltpu.async_copy(bufs[r], acc.at[dst_v.at[cur]], ssems[r],
                                 add=True)
            for r in range(NBUF):
                nxt = j + NBUF + r

                @pl.when(nxt < IBLK)
                def _():
                    pltpu.make_async_copy(bufs[r], acc.at[dst_v.at[r]],
                                          ssems[r]).wait()
                    pltpu.async_copy(g_hbm.at[src_v.at[nxt]], bufs[r],
                                     gsems[r])

        for r in range(NBUF):
            pltpu.make_async_copy(bufs[r], acc.at[dst_v.at[r]],
                                  ssems[r]).wait()

    plsc.subcore_barrier()

    @pl.loop(0, RPW // CH_E)
    def _wb(i):
        pltpu.async_copy(acc.at[pl.ds(s * RPW + i * CH_E, CH_E)],
                         out_hbm.at[c, pl.ds(s * RPW + i * CH_E, CH_E)],
                         gsems[0])

    @pl.loop(0, RPW // CH_E)
    def _wbw(i):
        pltpu.make_async_copy(acc.at[pl.ds(s * RPW, CH_E)],
                              out_hbm.at[c, pl.ds(s * RPW, CH_E)],
                              gsems[0]).wait()


def _dinv(deg_ref):
    deg = deg_ref[0, :, 0:1] + deg_ref[1, :, 0:1] + 1.0
    return lax.rsqrt(deg)


def _spec_rows(D):
    return pl.BlockSpec((RB, D), lambda i: (i, 0))


def _spec_parts(D):
    return pl.BlockSpec((NC, RB, D), lambda i: (0, i, 0))


def _spec_full(shape):
    return pl.BlockSpec(shape, lambda i: tuple(0 for _ in shape))


def _tc_call(body, in_specs, out_dim):
    return pl.pallas_call(
        body,
        grid=(NP // RB,),
        in_specs=in_specs,
        out_specs=_spec_rows(out_dim),
        out_shape=jax.ShapeDtypeStruct((NP, out_dim), jnp.float32),
    )


def _t_scale_mm(degp, z, W):
    di, do = W.shape

    def body(deg_ref, z_ref, w_ref, o_ref):
        o_ref[...] = _dinv(deg_ref) * jnp.dot(
            z_ref[...], w_ref[...], preferred_element_type=jnp.float32)

    return _tc_call(body, [_spec_parts(64), _spec_rows(di), _spec_full((di, do))],
                    do)(degp, z, W)


def _t_comb_mm(degp, S, g, b, W):
    di, do = W.shape

    def body(deg_ref, s_ref, g_ref, b_ref, w_ref, o_ref):
        dinv = _dinv(deg_ref)
        z = jnp.maximum(dinv * (s_ref[0] + s_ref[1] + g_ref[...]) + b_ref[...],
                        0.0)
        o_ref[...] = dinv * jnp.dot(z, w_ref[...],
                                    preferred_element_type=jnp.float32)

    return _tc_call(body, [_spec_parts(64), _spec_parts(di), _spec_rows(di),
                           _spec_full((1, di)), _spec_full((di, do))],
                    do)(degp, S, g, b.reshape(1, di), W)


def _t_comb_scale(degp, S, g, b):
    d = g.shape[1]

    def body(deg_ref, s_ref, g_ref, b_ref, o_ref):
        dinv = _dinv(deg_ref)
        z = jnp.maximum(dinv * (s_ref[0] + s_ref[1] + g_ref[...]) + b_ref[...],
                        0.0)
        o_ref[...] = dinv * z

    return _tc_call(body, [_spec_parts(64), _spec_parts(d), _spec_rows(d),
                           _spec_full((1, d))], d)(degp, S, g, b.reshape(1, d))


def _t_mm_post(degp, S, u, W, b):
    di, do = W.shape

    def body(deg_ref, s_ref, u_ref, w_ref, b_ref, o_ref):
        dinv = _dinv(deg_ref)
        m = dinv * (s_ref[0] + s_ref[1] + u_ref[...])
        z = jnp.maximum(jnp.dot(m, w_ref[...],
                                preferred_element_type=jnp.float32) + b_ref[...],
                        0.0)
        o_ref[...] = dinv * z

    return _tc_call(body, [_spec_parts(64), _spec_parts(di), _spec_rows(di),
                           _spec_full((di, do)), _spec_full((1, do))],
                    do)(degp, S, u, W, b.reshape(1, do))


def _t_mm2_post(degp, S, u, W, b, W2):
    di, dm = W.shape
    do = W2.shape[1]

    def body(deg_ref, s_ref, u_ref, w_ref, b_ref, w2_ref, o_ref):
        dinv = _dinv(deg_ref)
        m = dinv * (s_ref[0] + s_ref[1] + u_ref[...])
        z = jnp.maximum(jnp.dot(m, w_ref[...],
                                preferred_element_type=jnp.float32) + b_ref[...],
                        0.0)
        o_ref[...] = dinv * jnp.dot(z, w2_ref[...],
                                    preferred_element_type=jnp.float32)

    return _tc_call(body, [_spec_parts(64), _spec_parts(di), _spec_rows(di),
                           _spec_full((di, dm)), _spec_full((1, dm)),
                           _spec_full((dm, do))],
                    do)(degp, S, u, W, b.reshape(1, dm), W2)


def _t_final(degp, S, g, b):
    d = g.shape[1]

    def body(deg_ref, s_ref, g_ref, b_ref, o_ref):
        dinv = _dinv(deg_ref)
        o_ref[...] = dinv * (s_ref[0] + s_ref[1] + g_ref[...]) + b_ref[...]

    return _tc_call(body, [_spec_parts(64), _spec_parts(d), _spec_rows(d),
                           _spec_full((1, d))], d)(degp, S, g, b.reshape(1, d))


def kernel(x, edge_index, W1, b1, W2, b2, W3, b3, Wu3, bu3, Wu4, bu4, Wu5, bu5):
    src = edge_index[0].astype(jnp.int32)
    dst = edge_index[1].astype(jnp.int32)
    pad = NW * EPW - E
    fill = jnp.full((pad,), DUMMY, jnp.int32)
    srcw = jnp.concatenate([src, fill]).reshape(NW * (EPW // CH_E), CH_E)
    dstw = jnp.concatenate([dst, fill]).reshape(NW * (EPW // CH_E), CH_E)
    x_p = jnp.pad(x, ((0, NP - N), (0, 0)))

    z64 = jnp.zeros((CH_E, 64), jnp.float32)
    z128 = jnp.zeros((CH_E, 128), jnp.float32)

    sc64k = _sc_scatter(64)
    sc128k = _sc_scatter(128)

    def sc64(g):
        return sc64k(g, srcw, dstw, z64)

    def sc128(g):
        return sc128k(g, srcw, dstw, z128)

    W3p = jnp.pad(W3, ((0, 0), (0, 32)))
    b3p = jnp.pad(b3, (0, 32))
    Wu3p = jnp.pad(Wu3, ((0, 32), (0, 0)))

    ones_tab = jnp.ones((NP, 64), jnp.float32)
    degp = sc64(ones_tab)

    g1 = _t_scale_mm(degp, x_p, W1)
    S1 = sc128(g1)
    g2 = _t_comb_mm(degp, S1, g1, b1, W2)
    S2 = sc64(g2)
    g3 = _t_comb_mm(degp, S2, g2, b2, W3p)
    S3 = sc64(g3)
    u4 = _t_comb_scale(degp, S3, g3, b3p)
    S4 = sc64(u4)
    u5 = _t_mm_post(degp, S4, u4, Wu3p, bu3)
    S5 = sc64(u5)
    g6 = _t_mm2_post(degp, S5, u5, Wu4, bu4, Wu5)
    S6 = sc128(g6)
    outp = _t_final(degp, S6, g6, bu5)
    return outp[:N]


# speedup vs baseline: 2.0436x; 2.0436x over previous
import functools

import jax
import jax.numpy as jnp
from jax import lax
from jax.experimental import pallas as pl
from jax.experimental.pallas import tpu as pltpu
from jax.experimental.pallas import tpu_sc as plsc

N = 10000
E = 320000
NC, NS = 2, 16
NW = NC * NS
CH_E = 64
EPW = 10240
RPW = 640
NP = NS * RPW
DUMMY = N
IBLK = 40
NBUF = 4
RB = 1024


def _mesh():
    return plsc.VectorSubcoreMesh(core_axis_name="c", subcore_axis_name="s")


def _sc_scatter():
    @functools.partial(
        pl.kernel,
        out_type=jax.ShapeDtypeStruct((NC, NP, 64), jnp.float32),
        mesh=_mesh(),
        compiler_params=pltpu.CompilerParams(use_tc_tiling_on_sc=False),
        scratch_types=[
            pltpu.VMEM_SHARED((NP, 64), jnp.float32),
            pltpu.VMEM_SHARED((NP, 64), jnp.float32),
        ],
    )
    def k(g_hbm, srcw_hbm, dstw_hbm, zrow_hbm, out_hbm, acc, table):
        pl.run_scoped(
            functools.partial(_scatter_body, g_hbm, srcw_hbm, dstw_hbm,
                              zrow_hbm, out_hbm, acc, table),
            pltpu.VMEM((IBLK, CH_E), jnp.int32),
            pltpu.VMEM((IBLK, CH_E), jnp.int32),
            [pltpu.VMEM((CH_E, 64), jnp.float32) for _ in range(NBUF)],
            [pltpu.SemaphoreType.DMA for _ in range(NBUF)],
            [pltpu.SemaphoreType.DMA for _ in range(NBUF)],
        )

    return k


def _scatter_body(g_hbm, srcw_hbm, dstw_hbm, zrow_hbm, out_hbm, acc, table,
                  src_v, dst_v, bufs, gsems, ssems):
    chunks = EPW // CH_E
    c = lax.axis_index("c")
    s = lax.axis_index("s")
    w = s * NC + c
    pltpu.sync_copy(g_hbm.at[pl.ds(s * RPW, RPW)], table.at[pl.ds(s * RPW, RPW)])
    pltpu.sync_copy(zrow_hbm, bufs[0])

    @pl.loop(0, RPW // CH_E)
    def _zero(i):
        pltpu.sync_copy(bufs[0], acc.at[pl.ds(s * RPW + i * CH_E, CH_E)])

    plsc.subcore_barrier()

    @pl.loop(0, chunks // IBLK)
    def _block(kb):
        base = w * chunks + kb * IBLK
        pltpu.sync_copy(srcw_hbm.at[pl.ds(base, IBLK)], src_v)
        pltpu.sync_copy(dstw_hbm.at[pl.ds(base, IBLK)], dst_v)
        for r in range(NBUF):
            pltpu.async_copy(table.at[src_v.at[r]], bufs[r], gsems[r])

        @pl.loop(0, IBLK, step=NBUF)
        def _chunk(j):
            for r in range(NBUF):
                cur = j + r
                pltpu.make_async_copy(table.at[src_v.at[cur]], bufs[r],
                                      gsems[r]).wait()
                pltpu.async_copy(bufs[r], acc.at[dst_v.at[cur]], ssems[r],
                                 add=True)
            for r in range(NBUF):
                nxt = j + NBUF + r

                @pl.when(nxt < IBLK)
                def _():
                    pltpu.make_async_copy(bufs[r], acc.at[dst_v.at[r]],
                                          ssems[r]).wait()
                    pltpu.async_copy(table.at[src_v.at[nxt]], bufs[r],
                                     gsems[r])

        for r in range(NBUF):
            pltpu.make_async_copy(bufs[r], acc.at[dst_v.at[r]],
                                  ssems[r]).wait()

    plsc.subcore_barrier()

    @pl.loop(0, RPW // CH_E)
    def _wb(i):
        pltpu.async_copy(acc.at[pl.ds(s * RPW + i * CH_E, CH_E)],
                         out_hbm.at[c, pl.ds(s * RPW + i * CH_E, CH_E)],
                         gsems[0])

    @pl.loop(0, RPW // CH_E)
    def _wbw(i):
        pltpu.make_async_copy(acc.at[pl.ds(s * RPW, CH_E)],
                              out_hbm.at[c, pl.ds(s * RPW, CH_E)],
                              gsems[0]).wait()


def _dinv(deg_ref):
    deg = deg_ref[0, :, 0:1] + deg_ref[1, :, 0:1] + 1.0
    return lax.rsqrt(deg)


def _spec_rows(D):
    return pl.BlockSpec((RB, D), lambda i: (i, 0))


def _spec_parts(D):
    return pl.BlockSpec((NC, RB, D), lambda i: (0, i, 0))


def _spec_full(shape):
    return pl.BlockSpec(shape, lambda i: tuple(0 for _ in shape))


def _tc_call(body, in_specs, out_dim):
    return pl.pallas_call(
        body,
        grid=(NP // RB,),
        in_specs=in_specs,
        out_specs=_spec_rows(out_dim),
        out_shape=jax.ShapeDtypeStruct((NP, out_dim), jnp.float32),
    )


def _t_scale_mm(degp, z, W):
    di, do = W.shape

    def body(deg_ref, z_ref, w_ref, o_ref):
        o_ref[...] = _dinv(deg_ref) * jnp.dot(
            z_ref[...], w_ref[...], preferred_element_type=jnp.float32)

    return _tc_call(body, [_spec_parts(64), _spec_rows(di), _spec_full((di, do))],
                    do)(degp, z, W)


def _t_comb_mm(degp, S, g, b, W):
    di, do = W.shape

    def body(deg_ref, s_ref, g_ref, b_ref, w_ref, o_ref):
        dinv = _dinv(deg_ref)
        z = jnp.maximum(dinv * (s_ref[0] + s_ref[1] + g_ref[...]) + b_ref[...],
                        0.0)
        o_ref[...] = dinv * jnp.dot(z, w_ref[...],
                                    preferred_element_type=jnp.float32)

    return _tc_call(body, [_spec_parts(64), _spec_parts(di), _spec_rows(di),
                           _spec_full((1, di)), _spec_full((di, do))],
                    do)(degp, S, g, b.reshape(1, di), W)


def _t_comb_scale(degp, S, g, b):
    d = g.shape[1]

    def body(deg_ref, s_ref, g_ref, b_ref, o_ref):
        dinv = _dinv(deg_ref)
        z = jnp.maximum(dinv * (s_ref[0] + s_ref[1] + g_ref[...]) + b_ref[...],
                        0.0)
        o_ref[...] = dinv * z

    return _tc_call(body, [_spec_parts(64), _spec_parts(d), _spec_rows(d),
                           _spec_full((1, d))], d)(degp, S, g, b.reshape(1, d))


def _t_mm_post(degp, S, u, W, b):
    di, do = W.shape

    def body(deg_ref, s_ref, u_ref, w_ref, b_ref, o_ref):
        dinv = _dinv(deg_ref)
        m = dinv * (s_ref[0] + s_ref[1] + u_ref[...])
        z = jnp.maximum(jnp.dot(m, w_ref[...],
                                preferred_element_type=jnp.float32) + b_ref[...],
                        0.0)
        o_ref[...] = dinv * z

    return _tc_call(body, [_spec_parts(64), _spec_parts(di), _spec_rows(di),
                           _spec_full((di, do)), _spec_full((1, do))],
                    do)(degp, S, u, W, b.reshape(1, do))


def _t_mm2_post(degp, S, u, W, b, W2):
    di, dm = W.shape
    do = W2.shape[1]

    def body(deg_ref, s_ref, u_ref, w_ref, b_ref, w2_ref, o_ref):
        dinv = _dinv(deg_ref)
        m = dinv * (s_ref[0] + s_ref[1] + u_ref[...])
        z = jnp.maximum(jnp.dot(m, w_ref[...],
                                preferred_element_type=jnp.float32) + b_ref[...],
                        0.0)
        o_ref[...] = dinv * jnp.dot(z, w2_ref[...],
                                    preferred_element_type=jnp.float32)

    return _tc_call(body, [_spec_parts(64), _spec_parts(di), _spec_rows(di),
                           _spec_full((di, dm)), _spec_full((1, dm)),
                           _spec_full((dm, do))],
                    do)(degp, S, u, W, b.reshape(1, dm), W2)


def _t_final(degp, S, g, b):
    d = g.shape[1]

    def body(deg_ref, s_ref, g_ref, b_ref, o_ref):
        dinv = _dinv(deg_ref)
        o_ref[...] = dinv * (s_ref[0] + s_ref[1] + g_ref[...]) + b_ref[...]

    return _tc_call(body, [_spec_parts(64), _spec_parts(d), _spec_rows(d),
                           _spec_full((1, d))], d)(degp, S, g, b.reshape(1, d))


def kernel(x, edge_index, W1, b1, W2, b2, W3, b3, Wu3, bu3, Wu4, bu4, Wu5, bu5):
    src = edge_index[0].astype(jnp.int32)
    dst = edge_index[1].astype(jnp.int32)
    pad = NW * EPW - E
    fill = jnp.full((pad,), DUMMY, jnp.int32)
    srcw = jnp.concatenate([src, fill]).reshape(NW * (EPW // CH_E), CH_E)
    dstw = jnp.concatenate([dst, fill]).reshape(NW * (EPW // CH_E), CH_E)
    x_p = jnp.pad(x, ((0, NP - N), (0, 0)))

    z64 = jnp.zeros((CH_E, 64), jnp.float32)

    sc64k = _sc_scatter()

    def sc64(g):
        return sc64k(g, srcw, dstw, z64)

    def sc128(g):
        a = sc64(g[:, :64])
        b = sc64(g[:, 64:])
        return jnp.concatenate([a, b], axis=2)

    W3p = jnp.pad(W3, ((0, 0), (0, 32)))
    b3p = jnp.pad(b3, (0, 32))
    Wu3p = jnp.pad(Wu3, ((0, 32), (0, 0)))

    ones_tab = jnp.ones((NP, 64), jnp.float32)
    degp = sc64(ones_tab)

    g1 = _t_scale_mm(degp, x_p, W1)
    S1 = sc128(g1)
    g2 = _t_comb_mm(degp, S1, g1, b1, W2)
    S2 = sc64(g2)
    g3 = _t_comb_mm(degp, S2, g2, b2, W3p)
    S3 = sc64(g3)
    u4 = _t_comb_scale(degp, S3, g3, b3p)
    S4 = sc64(u4)
    u5 = _t_mm_post(degp, S4, u4, Wu3p, bu3)
    S5 = sc64(u5)
    g6 = _t_mm2_post(degp, S5, u5, Wu4, bu4, Wu5)
    S6 = sc128(g6)
    outp = _t_final(degp, S6, g6, bu5)
    return outp[:N]


# speedup vs baseline: 2.3182x; 1.1344x over previous
import functools

import jax
import jax.numpy as jnp
from jax import lax
from jax.experimental import pallas as pl
from jax.experimental.pallas import tpu as pltpu
from jax.experimental.pallas import tpu_sc as plsc

N = 10000
E = 320000
NC, NS = 2, 16
NW = NC * NS
CH_E = 64
EPW = 10240
RPW = 640
NP = NS * RPW
DUMMY = N
IBLK = 40
NBUF = 4
RB = 1024


def _mesh():
    return plsc.VectorSubcoreMesh(core_axis_name="c", subcore_axis_name="s")


def _sc_scatter(D):
    @functools.partial(
        pl.kernel,
        out_type=jax.ShapeDtypeStruct((NC, NP, D), jnp.float32),
        mesh=_mesh(),
        compiler_params=pltpu.CompilerParams(use_tc_tiling_on_sc=False),
        scratch_types=[
            pltpu.VMEM_SHARED((NP, D), jnp.float32),
            pltpu.VMEM_SHARED((NP, D), jnp.float32),
        ],
    )
    def k(g_hbm, srcw_hbm, dstw_hbm, zrow_hbm, out_hbm, acc, table):
        pl.run_scoped(
            functools.partial(_scatter_body, g_hbm, srcw_hbm, dstw_hbm,
                              zrow_hbm, out_hbm, acc, table),
            pltpu.VMEM((IBLK, CH_E), jnp.int32),
            pltpu.VMEM((IBLK, CH_E), jnp.int32),
            [pltpu.VMEM((CH_E, D), jnp.float32) for _ in range(NBUF)],
            [pltpu.SemaphoreType.DMA for _ in range(NBUF)],
            [pltpu.SemaphoreType.DMA for _ in range(NBUF)],
        )

    return k


def _scatter_body(g_hbm, srcw_hbm, dstw_hbm, zrow_hbm, out_hbm, acc, table,
                  src_v, dst_v, bufs, gsems, ssems):
    chunks = EPW // CH_E
    c = lax.axis_index("c")
    s = lax.axis_index("s")
    w = s * NC + c
    pltpu.sync_copy(g_hbm.at[pl.ds(s * RPW, RPW)], table.at[pl.ds(s * RPW, RPW)])
    pltpu.sync_copy(zrow_hbm, bufs[0])

    @pl.loop(0, RPW // CH_E)
    def _zero(i):
        pltpu.sync_copy(bufs[0], acc.at[pl.ds(s * RPW + i * CH_E, CH_E)])

    plsc.subcore_barrier()

    @pl.loop(0, chunks // IBLK)
    def _block(kb):
        base = w * chunks + kb * IBLK
        pltpu.sync_copy(srcw_hbm.at[pl.ds(base, IBLK)], src_v)
        pltpu.sync_copy(dstw_hbm.at[pl.ds(base, IBLK)], dst_v)
        for r in range(NBUF):
            pltpu.async_copy(table.at[src_v.at[r]], bufs[r], gsems[r])

        @pl.loop(0, IBLK, step=NBUF)
        def _chunk(j):
            for r in range(NBUF):
                cur = j + r
                pltpu.make_async_copy(table.at[src_v.at[cur]], bufs[r],
                                      gsems[r]).wait()
                pltpu.async_copy(bufs[r], acc.at[dst_v.at[cur]], ssems[r],
                                 add=True)
            for r in range(NBUF):
                nxt = j + NBUF + r

                @pl.when(nxt < IBLK)
                def _():
                    pltpu.make_async_copy(bufs[r], acc.at[dst_v.at[r]],
                                          ssems[r]).wait()
                    pltpu.async_copy(table.at[src_v.at[nxt]], bufs[r],
                                     gsems[r])

        for r in range(NBUF):
            pltpu.make_async_copy(bufs[r], acc.at[dst_v.at[r]],
                                  ssems[r]).wait()

    plsc.subcore_barrier()

    @pl.loop(0, RPW // CH_E)
    def _wb(i):
        pltpu.async_copy(acc.at[pl.ds(s * RPW + i * CH_E, CH_E)],
                         out_hbm.at[c, pl.ds(s * RPW + i * CH_E, CH_E)],
                         gsems[0])

    @pl.loop(0, RPW // CH_E)
    def _wbw(i):
        pltpu.make_async_copy(acc.at[pl.ds(s * RPW, CH_E)],
                              out_hbm.at[c, pl.ds(s * RPW, CH_E)],
                              gsems[0]).wait()


def _dinv(deg_ref):
    deg = deg_ref[0, :, 0:1] + deg_ref[1, :, 0:1] + 1.0
    return lax.rsqrt(deg)


def _spec_rows(D):
    return pl.BlockSpec((RB, D), lambda i: (i, 0))


def _spec_parts(D):
    return pl.BlockSpec((NC, RB, D), lambda i: (0, i, 0))


def _spec_full(shape):
    return pl.BlockSpec(shape, lambda i: tuple(0 for _ in shape))


def _tc_call(body, in_specs, out_dim):
    return pl.pallas_call(
        body,
        grid=(NP // RB,),
        in_specs=in_specs,
        out_specs=_spec_rows(out_dim),
        out_shape=jax.ShapeDtypeStruct((NP, out_dim), jnp.float32),
    )


def _t_scale_mm(degp, z, W):
    di, do = W.shape
    pd = degp.shape[2]

    def body(deg_ref, z_ref, w_ref, o_ref):
        o_ref[...] = _dinv(deg_ref) * jnp.dot(
            z_ref[...], w_ref[...], preferred_element_type=jnp.float32)

    return _tc_call(body, [_spec_parts(pd), _spec_rows(di), _spec_full((di, do))],
                    do)(degp, z, W)


def _t_comb_mm(degp, S, g, b, W):
    di, do = W.shape

    def body(deg_ref, s_ref, g_ref, b_ref, w_ref, o_ref):
        dinv = _dinv(deg_ref)
        z = jnp.maximum(dinv * (s_ref[0] + s_ref[1] + g_ref[...]) + b_ref[...],
                        0.0)
        o_ref[...] = dinv * jnp.dot(z, w_ref[...],
                                    preferred_element_type=jnp.float32)

    return _tc_call(body, [_spec_parts(degp.shape[2]), _spec_parts(di),
                           _spec_rows(di), _spec_full((1, di)),
                           _spec_full((di, do))],
                    do)(degp, S, g, b.reshape(1, di), W)


def _t_comb_scale(degp, S, g, b):
    d = g.shape[1]

    def body(deg_ref, s_ref, g_ref, b_ref, o_ref):
        dinv = _dinv(deg_ref)
        z = jnp.maximum(dinv * (s_ref[0] + s_ref[1] + g_ref[...]) + b_ref[...],
                        0.0)
        o_ref[...] = dinv * z

    return _tc_call(body, [_spec_parts(degp.shape[2]), _spec_parts(d),
                           _spec_rows(d), _spec_full((1, d))],
                    d)(degp, S, g, b.reshape(1, d))


def _t_mm_post(degp, S, u, W, b):
    di, do = W.shape

    def body(deg_ref, s_ref, u_ref, w_ref, b_ref, o_ref):
        dinv = _dinv(deg_ref)
        m = dinv * (s_ref[0] + s_ref[1] + u_ref[...])
        z = jnp.maximum(jnp.dot(m, w_ref[...],
                                preferred_element_type=jnp.float32) + b_ref[...],
                        0.0)
        o_ref[...] = dinv * z

    return _tc_call(body, [_spec_parts(degp.shape[2]), _spec_parts(di),
                           _spec_rows(di), _spec_full((di, do)),
                           _spec_full((1, do))],
                    do)(degp, S, u, W, b.reshape(1, do))


def _t_mm2_post(degp, S, u, W, b, W2):
    di, dm = W.shape
    do = W2.shape[1]

    def body(deg_ref, s_ref, u_ref, w_ref, b_ref, w2_ref, o_ref):
        dinv = _dinv(deg_ref)
        m = dinv * (s_ref[0] + s_ref[1] + u_ref[...])
        z = jnp.maximum(jnp.dot(m, w_ref[...],
                                preferred_element_type=jnp.float32) + b_ref[...],
                        0.0)
        o_ref[...] = dinv * jnp.dot(z, w2_ref[...],
                                    preferred_element_type=jnp.float32)

    return _tc_call(body, [_spec_parts(degp.shape[2]), _spec_parts(di),
                           _spec_rows(di), _spec_full((di, dm)),
                           _spec_full((1, dm)), _spec_full((dm, do))],
                    do)(degp, S, u, W, b.reshape(1, dm), W2)


def _t_final(degp, S, g, b):
    d = g.shape[1]

    def body(deg_ref, s_ref, g_ref, b_ref, o_ref):
        dinv = _dinv(deg_ref)
        o_ref[...] = dinv * (s_ref[0] + s_ref[1] + g_ref[...]) + b_ref[...]

    return _tc_call(body, [_spec_parts(degp.shape[2]), _spec_parts(d),
                           _spec_rows(d), _spec_full((1, d))],
                    d)(degp, S, g, b.reshape(1, d))


def kernel(x, edge_index, W1, b1, W2, b2, W3, b3, Wu3, bu3, Wu4, bu4, Wu5, bu5):
    src = edge_index[0].astype(jnp.int32)
    dst = edge_index[1].astype(jnp.int32)
    pad = NW * EPW - E
    fill = jnp.full((pad,), DUMMY, jnp.int32)
    srcw = jnp.concatenate([src, fill]).reshape(NW * (EPW // CH_E), CH_E)
    dstw = jnp.concatenate([dst, fill]).reshape(NW * (EPW // CH_E), CH_E)
    x_p = jnp.pad(x, ((0, NP - N), (0, 0)))

    z64 = jnp.zeros((CH_E, 64), jnp.float32)
    z32 = jnp.zeros((CH_E, 32), jnp.float32)

    sc64k = _sc_scatter(64)
    sc32k = _sc_scatter(32)

    def sc64(g):
        return sc64k(g, srcw, dstw, z64)

    def sc32(g):
        return sc32k(g, srcw, dstw, z32)

    def sc128(g):
        a = sc64(g[:, :64])
        b = sc64(g[:, 64:])
        return jnp.concatenate([a, b], axis=2)

    ones_tab = jnp.ones((NP, 32), jnp.float32)
    degp = sc32(ones_tab)

    g1 = _t_scale_mm(degp, x_p, W1)
    S1 = sc128(g1)
    g2 = _t_comb_mm(degp, S1, g1, b1, W2)
    S2 = sc64(g2)
    g3 = _t_comb_mm(degp, S2, g2, b2, W3)
    S3 = sc32(g3)
    u4 = _t_comb_scale(degp, S3, g3, b3)
    S4 = sc32(u4)
    u5 = _t_mm_post(degp, S4, u4, Wu3, bu3)
    S5 = sc64(u5)
    g6 = _t_mm2_post(degp, S5, u5, Wu4, bu4, Wu5)
    S6 = sc128(g6)
    outp = _t_final(degp, S6, g6, bu5)
    return outp[:N]


# speedup vs baseline: 2.5047x; 1.0805x over previous
import functools

import jax
import jax.numpy as jnp
from jax import lax
from jax.experimental import pallas as pl
from jax.experimental.pallas import tpu as pltpu
from jax.experimental.pallas import tpu_sc as plsc

N = 10000
E = 320000
NC, NS = 2, 16
NW = NC * NS
CH_E = 64
EPW = 10240
RPW = 640
NP = NS * RPW
DUMMY = N
IBLK = 40
NBUF = 4
RB = 1024


def _mesh():
    return plsc.VectorSubcoreMesh(core_axis_name="c", subcore_axis_name="s")


def _sc_scatter(D, P):
    @functools.partial(
        pl.kernel,
        out_type=jax.ShapeDtypeStruct((NC, NP, P * D), jnp.float32),
        mesh=_mesh(),
        compiler_params=pltpu.CompilerParams(use_tc_tiling_on_sc=False),
        scratch_types=[
            pltpu.VMEM_SHARED((NP, D), jnp.float32),
            pltpu.VMEM_SHARED((NP, D), jnp.float32),
        ],
    )
    def k(g_hbm, srcw_hbm, dstw_hbm, zrow_hbm, out_hbm, acc, table):
        pl.run_scoped(
            functools.partial(_scatter_body, D, P, g_hbm, srcw_hbm, dstw_hbm,
                              zrow_hbm, out_hbm, acc, table),
            pltpu.VMEM((IBLK, CH_E), jnp.int32),
            pltpu.VMEM((IBLK, CH_E), jnp.int32),
            [pltpu.VMEM((CH_E, D), jnp.float32) for _ in range(NBUF)],
            [pltpu.SemaphoreType.DMA for _ in range(NBUF)],
            [pltpu.SemaphoreType.DMA for _ in range(NBUF)],
        )

    return k


def _scatter_body(D, P, g_hbm, srcw_hbm, dstw_hbm, zrow_hbm, out_hbm, acc,
                  table, src_v, dst_v, bufs, gsems, ssems):
    chunks = EPW // CH_E
    c = lax.axis_index("c")
    s = lax.axis_index("s")
    w = s * NC + c

    for p in range(P):
        pltpu.sync_copy(g_hbm.at[pl.ds(s * RPW, RPW), pl.ds(p * D, D)],
                        table.at[pl.ds(s * RPW, RPW)])
        pltpu.sync_copy(zrow_hbm, bufs[0])

        @pl.loop(0, RPW // CH_E)
        def _zero(i):
            pltpu.sync_copy(bufs[0], acc.at[pl.ds(s * RPW + i * CH_E, CH_E)])

        plsc.subcore_barrier()

        @pl.loop(0, chunks // IBLK)
        def _block(kb):
            base = w * chunks + kb * IBLK
            pltpu.sync_copy(srcw_hbm.at[pl.ds(base, IBLK)], src_v)
            pltpu.sync_copy(dstw_hbm.at[pl.ds(base, IBLK)], dst_v)
            for r in range(NBUF):
                pltpu.async_copy(table.at[src_v.at[r]], bufs[r], gsems[r])

            @pl.loop(0, IBLK, step=NBUF)
            def _chunk(j):
                for r in range(NBUF):
                    cur = j + r
                    pltpu.make_async_copy(table.at[src_v.at[cur]], bufs[r],
                                          gsems[r]).wait()
                    pltpu.async_copy(bufs[r], acc.at[dst_v.at[cur]], ssems[r],
                                     add=True)
                for r in range(NBUF):
                    nxt = j + NBUF + r

                    @pl.when(nxt < IBLK)
                    def _():
                        pltpu.make_async_copy(bufs[r], acc.at[dst_v.at[r]],
                                              ssems[r]).wait()
                        pltpu.async_copy(table.at[src_v.at[nxt]], bufs[r],
                                         gsems[r])

            for r in range(NBUF):
                pltpu.make_async_copy(bufs[r], acc.at[dst_v.at[r]],
                                      ssems[r]).wait()

        plsc.subcore_barrier()

        @pl.loop(0, RPW // CH_E)
        def _wb(i):
            pltpu.async_copy(
                acc.at[pl.ds(s * RPW + i * CH_E, CH_E)],
                out_hbm.at[c, pl.ds(s * RPW + i * CH_E, CH_E),
                           pl.ds(p * D, D)],
                gsems[0])

        @pl.loop(0, RPW // CH_E)
        def _wbw(i):
            pltpu.make_async_copy(
                acc.at[pl.ds(s * RPW, CH_E)],
                out_hbm.at[c, pl.ds(s * RPW, CH_E), pl.ds(p * D, D)],
                gsems[0]).wait()


def _dinv(deg_ref):
    deg = deg_ref[0, :, 0:1] + deg_ref[1, :, 0:1] + 1.0
    return lax.rsqrt(deg)


def _spec_rows(D):
    return pl.BlockSpec((RB, D), lambda i: (i, 0))


def _spec_parts(D):
    return pl.BlockSpec((NC, RB, D), lambda i: (0, i, 0))


def _spec_full(shape):
    return pl.BlockSpec(shape, lambda i: tuple(0 for _ in shape))


def _tc_call(body, in_specs, out_dim):
    return pl.pallas_call(
        body,
        grid=(NP // RB,),
        in_specs=in_specs,
        out_specs=_spec_rows(out_dim),
        out_shape=jax.ShapeDtypeStruct((NP, out_dim), jnp.float32),
    )


def _t_scale_mm(degp, z, W):
    di, do = W.shape
    pd = degp.shape[2]

    def body(deg_ref, z_ref, w_ref, o_ref):
        o_ref[...] = _dinv(deg_ref) * jnp.dot(
            z_ref[...], w_ref[...], preferred_element_type=jnp.float32)

    return _tc_call(body, [_spec_parts(pd), _spec_rows(di), _spec_full((di, do))],
                    do)(degp, z, W)


def _t_comb_mm(degp, S, g, b, W):
    di, do = W.shape

    def body(deg_ref, s_ref, g_ref, b_ref, w_ref, o_ref):
        dinv = _dinv(deg_ref)
        z = jnp.maximum(dinv * (s_ref[0] + s_ref[1] + g_ref[...]) + b_ref[...],
                        0.0)
        o_ref[...] = dinv * jnp.dot(z, w_ref[...],
                                    preferred_element_type=jnp.float32)

    return _tc_call(body, [_spec_parts(degp.shape[2]), _spec_parts(di),
                           _spec_rows(di), _spec_full((1, di)),
                           _spec_full((di, do))],
                    do)(degp, S, g, b.reshape(1, di), W)


def _t_comb_scale(degp, S, g, b):
    d = g.shape[1]

    def body(deg_ref, s_ref, g_ref, b_ref, o_ref):
        dinv = _dinv(deg_ref)
        z = jnp.maximum(dinv * (s_ref[0] + s_ref[1] + g_ref[...]) + b_ref[...],
                        0.0)
        o_ref[...] = dinv * z

    return _tc_call(body, [_spec_parts(degp.shape[2]), _spec_parts(d),
                           _spec_rows(d), _spec_full((1, d))],
                    d)(degp, S, g, b.reshape(1, d))


def _t_mm_post(degp, S, u, W, b):
    di, do = W.shape

    def body(deg_ref, s_ref, u_ref, w_ref, b_ref, o_ref):
        dinv = _dinv(deg_ref)
        m = dinv * (s_ref[0] + s_ref[1] + u_ref[...])
        z = jnp.maximum(jnp.dot(m, w_ref[...],
                                preferred_element_type=jnp.float32) + b_ref[...],
                        0.0)
        o_ref[...] = dinv * z

    return _tc_call(body, [_spec_parts(degp.shape[2]), _spec_parts(di),
                           _spec_rows(di), _spec_full((di, do)),
                           _spec_full((1, do))],
                    do)(degp, S, u, W, b.reshape(1, do))


def _t_mm2_post(degp, S, u, W, b, W2):
    di, dm = W.shape
    do = W2.shape[1]

    def body(deg_ref, s_ref, u_ref, w_ref, b_ref, w2_ref, o_ref):
        dinv = _dinv(deg_ref)
        m = dinv * (s_ref[0] + s_ref[1] + u_ref[...])
        z = jnp.maximum(jnp.dot(m, w_ref[...],
                                preferred_element_type=jnp.float32) + b_ref[...],
                        0.0)
        o_ref[...] = dinv * jnp.dot(z, w2_ref[...],
                                    preferred_element_type=jnp.float32)

    return _tc_call(body, [_spec_parts(degp.shape[2]), _spec_parts(di),
                           _spec_rows(di), _spec_full((di, dm)),
                           _spec_full((1, dm)), _spec_full((dm, do))],
                    do)(degp, S, u, W, b.reshape(1, dm), W2)


def _t_final(degp, S, g, b):
    d = g.shape[1]

    def body(deg_ref, s_ref, g_ref, b_ref, o_ref):
        dinv = _dinv(deg_ref)
        o_ref[...] = dinv * (s_ref[0] + s_ref[1] + g_ref[...]) + b_ref[...]

    return _tc_call(body, [_spec_parts(degp.shape[2]), _spec_parts(d),
                           _spec_rows(d), _spec_full((1, d))],
                    d)(degp, S, g, b.reshape(1, d))


def kernel(x, edge_index, W1, b1, W2, b2, W3, b3, Wu3, bu3, Wu4, bu4, Wu5, bu5):
    src = edge_index[0].astype(jnp.int32)
    dst = edge_index[1].astype(jnp.int32)
    pad = NW * EPW - E
    fill = jnp.full((pad,), DUMMY, jnp.int32)
    srcw = jnp.concatenate([src, fill]).reshape(NW * (EPW // CH_E), CH_E)
    dstw = jnp.concatenate([dst, fill]).reshape(NW * (EPW // CH_E), CH_E)
    x_p = jnp.pad(x, ((0, NP - N), (0, 0)))

    z64 = jnp.zeros((CH_E, 64), jnp.float32)
    z32 = jnp.zeros((CH_E, 32), jnp.float32)

    sc64k = _sc_scatter(64, 1)
    sc128k = _sc_scatter(64, 2)
    sc32k = _sc_scatter(32, 1)

    def sc64(g):
        return sc64k(g, srcw, dstw, z64)

    def sc32(g):
        return sc32k(g, srcw, dstw, z32)

    def sc128(g):
        return sc128k(g, srcw, dstw, z64)

    ones_tab = jnp.ones((NP, 32), jnp.float32)
    degp = sc32(ones_tab)

    g1 = _t_scale_mm(degp, x_p, W1)
    S1 = sc128(g1)
    g2 = _t_comb_mm(degp, S1, g1, b1, W2)
    S2 = sc64(g2)
    g3 = _t_comb_mm(degp, S2, g2, b2, W3)
    S3 = sc32(g3)
    u4 = _t_comb_scale(degp, S3, g3, b3)
    S4 = sc32(u4)
    u5 = _t_mm_post(degp, S4, u4, Wu3, bu3)
    S5 = sc64(u5)
    g6 = _t_mm2_post(degp, S5, u5, Wu4, bu4, Wu5)
    S6 = sc128(g6)
    outp = _t_final(degp, S6, g6, bu5)
    return outp[:N]


# speedup vs baseline: 2.5872x; 1.0329x over previous
import functools

import jax
import jax.numpy as jnp
from jax import lax
from jax.experimental import pallas as pl
from jax.experimental.pallas import tpu as pltpu
from jax.experimental.pallas import tpu_sc as plsc

N = 10000
E = 320000
NC, NS = 2, 16
NW = NC * NS
CH_E = 64
EPW = 10240
RPW = 640
NP = NS * RPW
DUMMY = N
IBLK = 40
NBUF = 4
RB = 1024


def _mesh():
    return plsc.VectorSubcoreMesh(core_axis_name="c", subcore_axis_name="s")


def _sc_scatter(D, P):
    @functools.partial(
        pl.kernel,
        out_type=jax.ShapeDtypeStruct((NC, NP, P * D), jnp.float32),
        mesh=_mesh(),
        compiler_params=pltpu.CompilerParams(use_tc_tiling_on_sc=False),
        scratch_types=[
            pltpu.VMEM_SHARED((NP, D), jnp.float32),
            pltpu.VMEM_SHARED((NP, D), jnp.float32),
        ],
    )
    def k(g_hbm, sdw_hbm, zrow_hbm, out_hbm, acc, table):
        pl.run_scoped(
            functools.partial(_scatter_body, D, P, g_hbm, sdw_hbm,
                              zrow_hbm, out_hbm, acc, table),
            pltpu.VMEM((2 * IBLK, CH_E), jnp.int32),
            [pltpu.VMEM((CH_E, D), jnp.float32) for _ in range(NBUF)],
            [pltpu.SemaphoreType.DMA for _ in range(NBUF)],
            [pltpu.SemaphoreType.DMA for _ in range(NBUF)],
        )

    return k


def _scatter_body(D, P, g_hbm, sdw_hbm, zrow_hbm, out_hbm, acc,
                  table, sd_v, bufs, gsems, ssems):
    chunks = EPW // CH_E
    blocks = chunks // IBLK
    c = lax.axis_index("c")
    s = lax.axis_index("s")
    w = s * NC + c

    for p in range(P):
        stage = pltpu.async_copy(
            g_hbm.at[pl.ds(s * RPW, RPW), pl.ds(p * D, D)],
            table.at[pl.ds(s * RPW, RPW)], gsems[1])
        pltpu.sync_copy(zrow_hbm, bufs[0])

        @pl.loop(0, RPW // CH_E)
        def _zero(i):
            pltpu.sync_copy(bufs[0], acc.at[pl.ds(s * RPW + i * CH_E, CH_E)])

        stage.wait()
        plsc.subcore_barrier()

        @pl.loop(0, blocks)
        def _block(kb):
            base = (w * blocks + kb) * 2 * IBLK
            pltpu.sync_copy(sdw_hbm.at[pl.ds(base, 2 * IBLK)], sd_v)
            for r in range(NBUF):
                pltpu.async_copy(table.at[sd_v.at[r]], bufs[r], gsems[r])

            @pl.loop(0, IBLK, step=NBUF)
            def _chunk(j):
                for r in range(NBUF):
                    cur = j + r
                    pltpu.make_async_copy(table.at[sd_v.at[cur]], bufs[r],
                                          gsems[r]).wait()
                    pltpu.async_copy(bufs[r], acc.at[sd_v.at[IBLK + cur]],
                                     ssems[r], add=True)
                for r in range(NBUF):
                    nxt = j + NBUF + r

                    @pl.when(nxt < IBLK)
                    def _():
                        pltpu.make_async_copy(bufs[r], acc.at[sd_v.at[r]],
                                              ssems[r]).wait()
                        pltpu.async_copy(table.at[sd_v.at[nxt]], bufs[r],
                                         gsems[r])

            for r in range(NBUF):
                pltpu.make_async_copy(bufs[r], acc.at[sd_v.at[r]],
                                      ssems[r]).wait()

        plsc.subcore_barrier()

        @pl.loop(0, RPW // CH_E)
        def _wb(i):
            pltpu.async_copy(
                acc.at[pl.ds(s * RPW + i * CH_E, CH_E)],
                out_hbm.at[c, pl.ds(s * RPW + i * CH_E, CH_E),
                           pl.ds(p * D, D)],
                gsems[0])

        @pl.loop(0, RPW // CH_E)
        def _wbw(i):
            pltpu.make_async_copy(
                acc.at[pl.ds(s * RPW, CH_E)],
                out_hbm.at[c, pl.ds(s * RPW, CH_E), pl.ds(p * D, D)],
                gsems[0]).wait()


def _dinv(deg_ref):
    deg = deg_ref[0, :, 0:1] + deg_ref[1, :, 0:1] + 1.0
    return lax.rsqrt(deg)


def _spec_rows(D):
    return pl.BlockSpec((RB, D), lambda i: (i, 0))


def _spec_parts(D):
    return pl.BlockSpec((NC, RB, D), lambda i: (0, i, 0))


def _spec_full(shape):
    return pl.BlockSpec(shape, lambda i: tuple(0 for _ in shape))


def _tc_call(body, in_specs, out_dim):
    return pl.pallas_call(
        body,
        grid=(NP // RB,),
        in_specs=in_specs,
        out_specs=_spec_rows(out_dim),
        out_shape=jax.ShapeDtypeStruct((NP, out_dim), jnp.float32),
    )


def _t_scale_mm(degp, z, W):
    di, do = W.shape
    pd = degp.shape[2]

    def body(deg_ref, z_ref, w_ref, o_ref):
        o_ref[...] = _dinv(deg_ref) * jnp.dot(
            z_ref[...], w_ref[...], preferred_element_type=jnp.float32)

    return _tc_call(body, [_spec_parts(pd), _spec_rows(di), _spec_full((di, do))],
                    do)(degp, z, W)


def _t_comb_mm(degp, S, g, b, W):
    di, do = W.shape

    def body(deg_ref, s_ref, g_ref, b_ref, w_ref, o_ref):
        dinv = _dinv(deg_ref)
        z = jnp.maximum(dinv * (s_ref[0] + s_ref[1] + g_ref[...]) + b_ref[...],
                        0.0)
        o_ref[...] = dinv * jnp.dot(z, w_ref[...],
                                    preferred_element_type=jnp.float32)

    return _tc_call(body, [_spec_parts(degp.shape[2]), _spec_parts(di),
                           _spec_rows(di), _spec_full((1, di)),
                           _spec_full((di, do))],
                    do)(degp, S, g, b.reshape(1, di), W)


def _t_comb_scale(degp, S, g, b):
    d = g.shape[1]

    def body(deg_ref, s_ref, g_ref, b_ref, o_ref):
        dinv = _dinv(deg_ref)
        z = jnp.maximum(dinv * (s_ref[0] + s_ref[1] + g_ref[...]) + b_ref[...],
                        0.0)
        o_ref[...] = dinv * z

    return _tc_call(body, [_spec_parts(degp.shape[2]), _spec_parts(d),
                           _spec_rows(d), _spec_full((1, d))],
                    d)(degp, S, g, b.reshape(1, d))


def _t_mm_post(degp, S, u, W, b):
    di, do = W.shape

    def body(deg_ref, s_ref, u_ref, w_ref, b_ref, o_ref):
        dinv = _dinv(deg_ref)
        m = dinv * (s_ref[0] + s_ref[1] + u_ref[...])
        z = jnp.maximum(jnp.dot(m, w_ref[...],
                                preferred_element_type=jnp.float32) + b_ref[...],
                        0.0)
        o_ref[...] = dinv * z

    return _tc_call(body, [_spec_parts(degp.shape[2]), _spec_parts(di),
                           _spec_rows(di), _spec_full((di, do)),
                           _spec_full((1, do))],
                    do)(degp, S, u, W, b.reshape(1, do))


def _t_mm2_post(degp, S, u, W, b, W2):
    di, dm = W.shape
    do = W2.shape[1]

    def body(deg_ref, s_ref, u_ref, w_ref, b_ref, w2_ref, o_ref):
        dinv = _dinv(deg_ref)
        m = dinv * (s_ref[0] + s_ref[1] + u_ref[...])
        z = jnp.maximum(jnp.dot(m, w_ref[...],
                                preferred_element_type=jnp.float32) + b_ref[...],
                        0.0)
        o_ref[...] = dinv * jnp.dot(z, w2_ref[...],
                                    preferred_element_type=jnp.float32)

    return _tc_call(body, [_spec_parts(degp.shape[2]), _spec_parts(di),
                           _spec_rows(di), _spec_full((di, dm)),
                           _spec_full((1, dm)), _spec_full((dm, do))],
                    do)(degp, S, u, W, b.reshape(1, dm), W2)


def _t_final(degp, S, g, b):
    d = g.shape[1]

    def body(deg_ref, s_ref, g_ref, b_ref, o_ref):
        dinv = _dinv(deg_ref)
        o_ref[...] = dinv * (s_ref[0] + s_ref[1] + g_ref[...]) + b_ref[...]

    return _tc_call(body, [_spec_parts(degp.shape[2]), _spec_parts(d),
                           _spec_rows(d), _spec_full((1, d))],
                    d)(degp, S, g, b.reshape(1, d))


def kernel(x, edge_index, W1, b1, W2, b2, W3, b3, Wu3, bu3, Wu4, bu4, Wu5, bu5):
    src = edge_index[0].astype(jnp.int32)
    dst = edge_index[1].astype(jnp.int32)
    pad = NW * EPW - E
    fill = jnp.full((pad,), DUMMY, jnp.int32)
    chunks = EPW // CH_E
    blocks = chunks // IBLK
    srcw = jnp.concatenate([src, fill]).reshape(NW * blocks, IBLK, CH_E)
    dstw = jnp.concatenate([dst, fill]).reshape(NW * blocks, IBLK, CH_E)
    sdw = jnp.stack([srcw, dstw], axis=1).reshape(NW * blocks * 2 * IBLK, CH_E)
    x_p = jnp.pad(x, ((0, NP - N), (0, 0)))

    z64 = jnp.zeros((CH_E, 64), jnp.float32)
    z32 = jnp.zeros((CH_E, 32), jnp.float32)

    sc64k = _sc_scatter(64, 1)
    sc128k = _sc_scatter(64, 2)
    sc32k = _sc_scatter(32, 1)

    def sc64(g):
        return sc64k(g, sdw, z64)

    def sc32(g):
        return sc32k(g, sdw, z32)

    def sc128(g):
        return sc128k(g, sdw, z64)

    ones_tab = jnp.ones((NP, 32), jnp.float32)
    degp = sc32(ones_tab)

    g1 = _t_scale_mm(degp, x_p, W1)
    S1 = sc128(g1)
    g2 = _t_comb_mm(degp, S1, g1, b1, W2)
    S2 = sc64(g2)
    g3 = _t_comb_mm(degp, S2, g2, b2, W3)
    S3 = sc32(g3)
    u4 = _t_comb_scale(degp, S3, g3, b3)
    S4 = sc32(u4)
    u5 = _t_mm_post(degp, S4, u4, Wu3, bu3)
    S5 = sc64(u5)
    g6 = _t_mm2_post(degp, S5, u5, Wu4, bu4, Wu5)
    S6 = sc128(g6)
    outp = _t_final(degp, S6, g6, bu5)
    return outp[:N]
